# baseline (speedup 1.0000x reference)
import functools
import jax
import jax.numpy as jnp
from jax import lax
from jax.experimental import pallas as pl
from jax.experimental.pallas import tpu as pltpu
from jax.experimental.pallas import tpu_sc as plsc

_CHUNK = 16
_UNROLL = 8


def kernel(x, weight):
    B, S, H = x.shape
    w = weight[:S]
    NW = 32
    total_rows = B * S
    rpw = total_rows // NW
    cw = _CHUNK * H
    nch = rpw // _CHUNK
    x1 = x.reshape(total_rows * H)
    w1 = w.reshape(S * H)
    mesh = plsc.VectorSubcoreMesh(core_axis_name="c", subcore_axis_name="s")

    vmem = pltpu.VMEM((cw,), jnp.float32)

    @functools.partial(
        pl.kernel,
        mesh=mesh,
        out_type=jax.ShapeDtypeStruct((total_rows * H,), jnp.float32),
        scratch_types=[
            vmem, vmem, vmem, vmem, vmem, vmem,
            pltpu.SemaphoreType.DMA,
            pltpu.SemaphoreType.DMA,
            pltpu.SemaphoreType.DMA,
            pltpu.SemaphoreType.DMA,
        ],
    )
    def run(x_hbm, w_hbm, o_hbm, xb0, xb1, wb0, wb1, ob0, ob1,
            si0, si1, so0, so1):
        c = lax.axis_index("c")
        s_idx = lax.axis_index("s")
        wid = s_idx * 2 + c
        row0 = wid * rpw
        xoff = row0 * H
        woff = (row0 % S) * H
        xbufs = (xb0, xb1)
        wbufs = (wb0, wb1)
        obufs = (ob0, ob1)
        sin = (si0, si1)
        sout = (so0, so1)

        def start_in(t, b):
            pltpu.async_copy(x_hbm.at[pl.ds(xoff + t * cw, cw)], xbufs[b], sin[b])
            pltpu.async_copy(w_hbm.at[pl.ds(woff + t * cw, cw)], wbufs[b], sin[b])

        def wait_in(t, b):
            pltpu.make_async_copy(
                x_hbm.at[pl.ds(xoff + t * cw, cw)], xbufs[b], sin[b]).wait()
            pltpu.make_async_copy(
                w_hbm.at[pl.ds(woff + t * cw, cw)], wbufs[b], sin[b]).wait()

        def start_out(t, b):
            pltpu.async_copy(obufs[b], o_hbm.at[pl.ds(xoff + t * cw, cw)], sout[b])

        def wait_out(t, b):
            pltpu.make_async_copy(
                obufs[b], o_hbm.at[pl.ds(xoff + t * cw, cw)], sout[b]).wait()

        def compute(b):
            xb, wb, ob = xbufs[b], wbufs[b], obufs[b]

            @plsc.parallel_loop(0, cw, 16, unroll=_UNROLL)
            def inner(i):
                sl = pl.ds(i, 16)
                ob[sl] = xb[sl] + wb[sl]

        start_in(0, 0)
        start_in(1, 1)

        def step(t2, carry):
            for b in range(2):
                t = t2 * 2 + b
                wait_in(t, b)
                compute(b)
                @pl.when(t2 > 0)
                def _():
                    wait_out(t - 2, b)
                start_out(t, b)
                @pl.when(t2 < nch // 2 - 1)
                def _():
                    start_in(t + 2, b)
            return carry

        lax.fori_loop(0, nch // 2, step, 0)
        wait_out(nch - 2, 0)
        wait_out(nch - 1, 1)

    out = run(x1, w1)
    return out.reshape(B, S, H)


# speedup vs baseline: 4.5632x; 4.5632x over previous
import jax
import jax.numpy as jnp
from jax.experimental import pallas as pl

_BS = 2048


def _add_kernel(x_ref, w_ref, o_ref):
    o_ref[...] = x_ref[...] + w_ref[...]


def kernel(x, weight):
    B, S, H = x.shape
    w = weight[:S]
    x2 = x.reshape(B * S, H)
    nsb = S // _BS
    out = pl.pallas_call(
        _add_kernel,
        grid=(nsb, B),
        in_specs=[
            pl.BlockSpec((_BS, H), lambda i, j: (j * nsb + i, 0)),
            pl.BlockSpec((_BS, H), lambda i, j: (i, 0)),
        ],
        out_specs=pl.BlockSpec((_BS, H), lambda i, j: (j * nsb + i, 0)),
        out_shape=jax.ShapeDtypeStruct((B * S, H), x.dtype),
    )(x2, w)
    return out.reshape(B, S, H)
